# Initial kernel scaffold; baseline (speedup 1.0000x reference)
import jax
import jax.numpy as jnp
from jax.experimental import pallas as pl


def kernel(x, weight):
    raise NotImplementedError("write your pallas kernel here")


# speedup vs baseline: 1.4919x; 1.4919x over previous
import functools

import jax
import jax.numpy as jnp
from jax import lax
from jax.experimental import pallas as pl
from jax.experimental.pallas import tpu as pltpu
from jax.experimental.pallas import tpu_sc as plsc

VOCAB = 1000000
EMB = 32
ROWS_TOTAL = 4096 * 200

NC = 2
NS = 16
NW = NC * NS
B_PER_W = ROWS_TOTAL // NW

GROUP = 128
GROUPS_PER_STEP = 5
STEP_ROWS = GROUP * GROUPS_PER_STEP
NBUF = 4
NSTEPS = B_PER_W // STEP_ROWS
NITER = NSTEPS // NBUF
IDX_ROWS = B_PER_W // GROUP

_mesh = plsc.VectorSubcoreMesh(core_axis_name="c", subcore_axis_name="s")


@functools.partial(
    pl.kernel,
    mesh=_mesh,
    out_type=jax.ShapeDtypeStruct((ROWS_TOTAL, EMB), jnp.float32),
    scratch_types=(
        [pltpu.VMEM((IDX_ROWS, GROUP), jnp.int32)]
        + [pltpu.VMEM((STEP_ROWS, EMB), jnp.float32) for _ in range(NBUF)]
        + [pltpu.SemaphoreType.DMA for _ in range(2 * NBUF)]
    ),
    compiler_params=pltpu.CompilerParams(use_tc_tiling_on_sc=False),
)
def _embed(x_hbm, table_hbm, out_hbm, idx_v, *bufs_and_sems):
    bufs = bufs_and_sems[:NBUF]
    gsems = bufs_and_sems[NBUF:2 * NBUF]
    wsems = bufs_and_sems[2 * NBUF:]

    wid = lax.axis_index("s") * NC + lax.axis_index("c")
    row_base = wid * B_PER_W

    pltpu.sync_copy(x_hbm.at[pl.ds(wid * IDX_ROWS, IDX_ROWS)], idx_v)

    def start_gathers(step, b):
        for j in range(GROUPS_PER_STEP):
            pltpu.async_copy(
                table_hbm.at[idx_v.at[step * GROUPS_PER_STEP + j]],
                bufs[b].at[pl.ds(j * GROUP, GROUP)],
                gsems[b],
            )

    def wait_gathers(b):
        pltpu.make_async_copy(
            table_hbm.at[pl.ds(0, STEP_ROWS)], bufs[b], gsems[b]
        ).wait()

    def start_write(step, b):
        pltpu.async_copy(
            bufs[b],
            out_hbm.at[pl.ds(row_base + step * STEP_ROWS, STEP_ROWS)],
            wsems[b],
        )

    def wait_write(b):
        pltpu.make_async_copy(
            table_hbm.at[pl.ds(0, STEP_ROWS)], bufs[b], wsems[b]
        ).wait()

    for b in range(NBUF):
        start_gathers(b, b)

    def body(t):
        for b in range(NBUF):
            wait_gathers(b)
            start_write(t * NBUF + b, b)
        for b in range(NBUF):
            wait_write(b)
            start_gathers((t + 1) * NBUF + b, b)

    pl.loop(0, NITER - 1)(body)

    t_last = NITER - 1
    for b in range(NBUF):
        wait_gathers(b)
        start_write(t_last * NBUF + b, b)
    for b in range(NBUF):
        wait_write(b)


def kernel(x, weight):
    x_flat = x.reshape(-1).astype(jnp.int32).reshape(NW * IDX_ROWS, GROUP)
    out = _embed(x_flat, weight)
    return out.reshape(x.shape + (EMB,))
